# baseline (speedup 1.0000x reference)
import jax
import jax.numpy as jnp
from jax import lax
from jax.experimental import pallas as pl
from jax.experimental.pallas import tpu as pltpu
from jax.experimental.pallas import tpu_sc as plsc

D_MODEL = 64
SCALE = 8.0
NC, NS, L = 2, 16, 16
NW = NC * NS
B_ROWS, S_LEN = 4096, 200
B = B_ROWS * S_LEN
BPW = B // NW
VOCAB = 1000000
CHUNK = 64
NCHUNK = BPW // CHUNK
NBUF = 5
LOOKAHEAD = 3
NSLC = D_MODEL // L
NVG = CHUNK // L


def _embed_body(x_hbm, table_hbm, out_hbm, idx_v, gbuf, obuf, *sems):
    gsems, ssems = sems[:NBUF], sems[NBUF:]
    wid = lax.axis_index("s") * NC + lax.axis_index("c")
    base = wid * BPW
    pltpu.sync_copy(x_hbm.at[pl.ds(base, BPW)], idx_v.at[pl.ds(0, BPW)])

    def slice_idx(c, s):
        return lax.shift_right_logical(
            idx_v[pl.ds(c * CHUNK + s * L, L)], 1)

    def gather_pair(c, s, b):
        return (table_hbm.at[slice_idx(c, s)],
                gbuf.at[b, pl.ds(s * L, L)], gsems[b])

    def scatter_pair(c, b):
        off = pl.multiple_of((base + c * CHUNK) // 2, CHUNK // 2)
        return (obuf.at[b],
                out_hbm.at[pl.ds(off, CHUNK // 2)],
                ssems[b])

    def fire_gather(c, b):
        for s in range(NVG):
            pltpu.async_copy(*gather_pair(c, s, b))

    def drain_gather(c, b):
        for s in range(NVG):
            pltpu.make_async_copy(*gather_pair(c, s, b)).wait()

    def select_scale(c, b):
        def row_body(row, carry):
            v = idx_v[pl.ds(c * CHUNK + row, L)]
            hb = (v[0] & 1) * D_MODEL
            row2 = row // 2
            ob = (row % 2) * D_MODEL
            for s in range(NSLC):
                obuf[b, row2, pl.ds(ob + s * L, L)] = (
                    gbuf[b, row, pl.ds(hb + s * L, L)] * SCALE)
            return carry

        lax.fori_loop(0, CHUNK, row_body, 0)

    def step(c, b):
        drain_gather(c, b)
        select_scale(c, b)
        pltpu.async_copy(*scatter_pair(c, b))
        f = c + LOOKAHEAD
        bf = (b + LOOKAHEAD) % NBUF

        @pl.when(c >= NBUF - LOOKAHEAD)
        def _():
            pltpu.make_async_copy(*scatter_pair(f - NBUF, bf)).wait()

        @pl.when(f < NCHUNK)
        def _():
            fire_gather(f, bf)

    for c in range(LOOKAHEAD):
        fire_gather(c, c)

    def outer(h, carry):
        c0 = h * NBUF
        for b in range(NBUF):
            step(c0 + b, b)
        return carry

    lax.fori_loop(0, NCHUNK // NBUF, outer, 0)

    for c in range(NCHUNK - (NBUF - LOOKAHEAD), NCHUNK):
        pltpu.make_async_copy(*scatter_pair(c, c % NBUF)).wait()


@jax.jit
def kernel(x, table):
    xf = x.reshape(B)
    table2 = table.reshape(VOCAB // 2, 2 * D_MODEL)
    out = pl.kernel(
        _embed_body,
        out_type=jax.ShapeDtypeStruct((B // 2, 2 * D_MODEL), jnp.float32),
        mesh=plsc.VectorSubcoreMesh(
            core_axis_name="c", subcore_axis_name="s",
            num_cores=NC, num_subcores=NS,
        ),
        scratch_types=(
            [pltpu.VMEM((BPW + L,), jnp.int32),
             pltpu.VMEM((NBUF, CHUNK, 2 * D_MODEL), jnp.float32),
             pltpu.VMEM((NBUF, CHUNK // 2, 2 * D_MODEL), jnp.float32)]
            + [pltpu.SemaphoreType.DMA] * (2 * NBUF)
        ),
        compiler_params=pltpu.CompilerParams(use_tc_tiling_on_sc=True),
    )(xf, table2)
    return out.reshape(B_ROWS, S_LEN, D_MODEL)


# speedup vs baseline: 1.5629x; 1.5629x over previous
import jax
import jax.numpy as jnp
from jax import lax
from jax.experimental import pallas as pl
from jax.experimental.pallas import tpu as pltpu
from jax.experimental.pallas import tpu_sc as plsc

D_MODEL = 64
SCALE = 8.0
NC, NS, L = 2, 16, 16
NW = NC * NS
B_ROWS, S_LEN = 4096, 200
B = B_ROWS * S_LEN
BPW = B // NW
CHUNK = 128
NCHUNK = BPW // CHUNK
NBUF = 8
LOOKAHEAD = 6
OUTER = NCHUNK // NBUF


def _embed_body(x_hbm, table_hbm, out_hbm, idx_v, rows_v, *sems):
    gsems, ssems = sems[:NBUF], sems[NBUF:]
    wid = lax.axis_index("s") * NC + lax.axis_index("c")
    base = wid * BPW
    pltpu.sync_copy(x_hbm.at[pl.ds(base, BPW)], idx_v)

    def gather_pair(c, b):
        return (table_hbm.at[idx_v.at[pl.ds(c * CHUNK, CHUNK)]],
                rows_v.at[b], gsems[b])

    def scatter_pair(c, b):
        return (rows_v.at[b], out_hbm.at[pl.ds(base + c * CHUNK, CHUNK)],
                ssems[b])

    def scale(b):
        def row_body(r, carry):
            for s in range(D_MODEL // L):
                sl = pl.ds(s * L, L)
                rows_v[b, r, sl] = rows_v[b, r, sl] * SCALE
            return carry

        lax.fori_loop(0, CHUNK, row_body, 0)

    def step(c, b, do_wait_ssem, do_fire):
        pltpu.make_async_copy(*gather_pair(c, b)).wait()
        scale(b)
        pltpu.async_copy(*scatter_pair(c, b))
        f = c + LOOKAHEAD
        bf = (b + LOOKAHEAD) % NBUF
        if do_wait_ssem:
            pltpu.make_async_copy(*scatter_pair(f - NBUF, bf)).wait()
        if do_fire:
            pltpu.async_copy(*gather_pair(f, bf))

    for c in range(LOOKAHEAD):
        pltpu.async_copy(*gather_pair(c, c))

    for b in range(NBUF):
        step(b, b, do_wait_ssem=(b + LOOKAHEAD >= NBUF), do_fire=True)

    def outer(h, carry):
        c0 = h * NBUF
        for b in range(NBUF):
            step(c0 + b, b, do_wait_ssem=True, do_fire=True)
        return carry

    lax.fori_loop(1, OUTER - 1, outer, 0)

    for b in range(NBUF):
        c = NCHUNK - NBUF + b
        step(c, b, do_wait_ssem=True, do_fire=(c + LOOKAHEAD < NCHUNK))

    for c in range(NCHUNK - (NBUF - LOOKAHEAD), NCHUNK):
        pltpu.make_async_copy(*scatter_pair(c, c % NBUF)).wait()


@jax.jit
def kernel(x, table):
    xf = x.reshape(B)
    out = pl.kernel(
        _embed_body,
        out_type=jax.ShapeDtypeStruct((B, D_MODEL), jnp.float32),
        mesh=plsc.VectorSubcoreMesh(
            core_axis_name="c", subcore_axis_name="s",
            num_cores=NC, num_subcores=NS,
        ),
        scratch_types=(
            [pltpu.VMEM((BPW,), jnp.int32),
             pltpu.VMEM((NBUF, CHUNK, D_MODEL), jnp.float32)]
            + [pltpu.SemaphoreType.DMA] * (2 * NBUF)
        ),
        compiler_params=pltpu.CompilerParams(use_tc_tiling_on_sc=False),
    )(xf, table)
    return out.reshape(B_ROWS, S_LEN, D_MODEL)
